# baseline (speedup 1.0000x reference)
import functools

import jax
import jax.numpy as jnp
from jax import lax
from jax.experimental import pallas as pl
from jax.experimental.pallas import tpu as pltpu
from jax.experimental.pallas import tpu_sc as plsc

N_LOC = 10000
NPAD = 10240
E = 160000
NG = 16
H = 64
GRU_H = 64
NC = 2
NS = 16
EPC = E // NS
SLC = NPAD // NS
CH_A = EPC // 16
KD = 80
CH_D = EPC // KD
NQ = 20
QN = NPAD // NQ
NB = 512
F32 = jnp.float32


def _proj_body(apply_elu, x_ref, w_ref, b_ref, as_ref, ad_ref, ab_ref,
               z_ref, sa_ref, da_ref, mx_ref):
    j = pl.program_id(1)
    x = x_ref[0]
    if apply_elu:
        x = jnp.where(x > 0, x, (jnp.exp(x) - 1.0))
    z = jnp.dot(x, w_ref[:], preferred_element_type=F32) + b_ref[:]
    z_ref[0] = jnp.concatenate([z, jnp.zeros((NB, 128 - H), F32)], axis=1)
    sa = jnp.sum(z * as_ref[:], axis=1)
    da = jnp.sum(z * ad_ref[:], axis=1) + ab_ref[0, 0]
    sa_ref[0, 0, 0] = sa
    da_ref[0, 0, 0] = da

    @pl.when(j == 0)
    def _():
        mx_ref[:] = jnp.full((1, 1, 128), -jnp.inf, F32)
    mx_ref[:] = jnp.maximum(mx_ref[:], jnp.max(sa))


def _make_proj(f_in, apply_elu):
    grid = (NG, NPAD // NB)
    return pl.pallas_call(
        functools.partial(_proj_body, apply_elu),
        grid=grid,
        in_specs=[
            pl.BlockSpec((1, NB, f_in), lambda i, j: (i, j, 0)),
            pl.BlockSpec((f_in, H), lambda i, j: (0, 0)),
            pl.BlockSpec((1, H), lambda i, j: (0, 0)),
            pl.BlockSpec((1, H), lambda i, j: (0, 0)),
            pl.BlockSpec((1, H), lambda i, j: (0, 0)),
            pl.BlockSpec((1, 1), lambda i, j: (0, 0)),
        ],
        out_specs=[
            pl.BlockSpec((1, NB, 128), lambda i, j: (i, j, 0)),
            pl.BlockSpec((1, 1, 1, NB), lambda i, j: (i, j, 0, 0)),
            pl.BlockSpec((1, 1, 1, NB), lambda i, j: (i, j, 0, 0)),
            pl.BlockSpec((1, 1, 128), lambda i, j: (i, 0, 0)),
        ],
        out_shape=[
            jax.ShapeDtypeStruct((NG, NPAD, 128), F32),
            jax.ShapeDtypeStruct((NG, NPAD // NB, 1, NB), F32),
            jax.ShapeDtypeStruct((NG, NPAD // NB, 1, NB), F32),
            jax.ShapeDtypeStruct((NG, 1, 128), F32),
        ],
    )


def _shift_body(da_ref, mx_ref, ca_ref):
    u = mx_ref[0, 0, 0] + da_ref[0, 0, 0]
    ca_ref[0, 0, 0] = jnp.where(u >= 0, u, 0.01 * u)


_shift = pl.pallas_call(
    _shift_body,
    grid=(NG, NPAD // NB),
    in_specs=[
        pl.BlockSpec((1, 1, 1, NB), lambda i, j: (i, j, 0, 0)),
        pl.BlockSpec((1, 1, 128), lambda i, j: (i, 0, 0)),
    ],
    out_specs=pl.BlockSpec((1, 1, 1, NB), lambda i, j: (i, j, 0, 0)),
    out_shape=jax.ShapeDtypeStruct((NG, NPAD // NB, 1, NB), F32),
)


def _gat_sc_body(z_hbm, sa_hbm, da_hbm, ca_hbm, src_hbm, dst_hbm,
                 out_hbm, den_hbm, rden_hbm,
                 sa_v, da_v, ca_v, den_v, rden_v, ex_v, src_v, dst_v, idxq_v,
                 red_v, rows_v, rows64_v, w_v,
                 o_sp, sem):
    cid = lax.axis_index("c")
    sid = lax.axis_index("s")
    ebase = sid * EPC
    nbase = sid * SLC
    nslice = pl.ds(nbase, SLC)

    pltpu.sync_copy(src_hbm.at[pl.ds(ebase, EPC)], src_v)
    pltpu.sync_copy(dst_hbm.at[pl.ds(ebase, EPC)], dst_v)

    zero16 = jnp.zeros((16,), F32)

    def per_graph(gl, _):
        g = cid * (NG // NC) + gl

        pltpu.sync_copy(sa_hbm.at[g], sa_v)
        pltpu.sync_copy(da_hbm.at[g], da_v)
        pltpu.sync_copy(ca_hbm.at[g], ca_v)

        def zden(i, _):
            den_v[pl.ds(i * 16, 16)] = zero16
            return 0
        lax.fori_loop(0, NPAD // 16, zden, 0)

        def edges_a(i, _):
            es = pl.ds(i * 16, 16)
            si = src_v[es]
            di = dst_v[es]
            s = plsc.load_gather(sa_v, [si])
            d = plsc.load_gather(da_v, [di])
            c = plsc.load_gather(ca_v, [di])
            u = s + d
            e = jnp.where(u >= 0, u, 0.01 * u)
            ex = jnp.exp(e - c)
            ex_v[es] = ex
            plsc.addupdate_scatter(den_v, [di], ex)
            return 0
        lax.fori_loop(0, CH_A, edges_a, 0)

        pltpu.sync_copy(den_v, den_hbm.at[cid, sid])
        plsc.subcore_barrier()

        for r in range(NS):
            pltpu.sync_copy(den_hbm.at[cid, r].at[nslice], red_v.at[r])

        def redc(c2, _):
            ls = pl.ds(c2 * 16, 16)
            acc = zero16

            def addr(r, a):
                return a + red_v[r, ls]
            acc = lax.fori_loop(0, NS, addr, acc)
            rec = 1.0 / acc
            rden_v[pl.ds(nbase + c2 * 16, 16)] = jnp.where(acc > 0, rec, 1.0)
            return 0
        lax.fori_loop(0, SLC // 16, redc, 0)
        pltpu.sync_copy(rden_v.at[nslice], rden_hbm.at[cid].at[nslice])
        plsc.subcore_barrier()
        pltpu.sync_copy(rden_hbm.at[cid], rden_v)

        def zrow(k, _):
            for q in range(4):
                rows64_v[k, pl.ds(q * 16, 16)] = zero16
            return 0

        for q in range(NQ):
            qb = q * QN
            lax.fori_loop(0, KD, zrow, 0)
            pltpu.sync_copy(rows64_v.at[pl.ds(0, QN // NS)],
                            o_sp.at[pl.ds(sid * (QN // NS), QN // NS)])
            plsc.subcore_barrier()

            def edges_d(j, _):
                base = j * KD
                for m in range(KD // 16):
                    es = pl.ds(base + m * 16, 16)
                    di = dst_v[es]
                    rd = plsc.load_gather(rden_v, [di])
                    w_v[pl.ds(m * 16, 16)] = ex_v[es] * rd
                    inr = (di >= qb) & (di < qb + QN)
                    idxq_v[0, pl.ds(m * 16, 16)] = jnp.where(inr, di - qb, QN)
                pltpu.async_copy(z_hbm.at[g].at[src_v.at[pl.ds(base, KD)]],
                                 rows_v, sem).wait()

                def srow(k, _):
                    wk = plsc.load_gather(
                        w_v, [jnp.full((16,), 0, jnp.int32) + k])
                    for qq in range(4):
                        ls = pl.ds(qq * 16, 16)
                        rows64_v[k, ls] = rows_v[k, ls] * wk
                    return 0
                lax.fori_loop(0, KD, srow, 0)
                pltpu.sync_copy(rows64_v, o_sp.at[idxq_v.at[0]], add=True)
                return 0
            lax.fori_loop(0, CH_D, edges_d, 0)
            plsc.subcore_barrier()

            pltpu.sync_copy(o_sp.at[pl.ds(sid * (QN // NS), QN // NS)],
                            rows64_v.at[pl.ds(0, QN // NS)])
            pltpu.sync_copy(rows64_v.at[pl.ds(0, QN // NS)],
                            out_hbm.at[g].at[pl.ds(qb + sid * (QN // NS),
                                                   QN // NS)])
            plsc.subcore_barrier()
        return 0

    lax.fori_loop(0, NG // NC, per_graph, 0)


_gat_sc = pl.kernel(
    _gat_sc_body,
    out_type=(jax.ShapeDtypeStruct((NG, NPAD, H), F32),
              jax.ShapeDtypeStruct((NC, NS, NPAD), F32),
              jax.ShapeDtypeStruct((NC, NPAD), F32)),
    mesh=plsc.VectorSubcoreMesh(core_axis_name="c", subcore_axis_name="s"),
    compiler_params=pltpu.CompilerParams(needs_layout_passes=False),
    scratch_types=[
        pltpu.VMEM((NPAD,), F32),
        pltpu.VMEM((NPAD,), F32),
        pltpu.VMEM((NPAD,), F32),
        pltpu.VMEM((NPAD,), F32),
        pltpu.VMEM((NPAD,), F32),
        pltpu.VMEM((EPC,), F32),
        pltpu.VMEM((EPC,), jnp.int32),
        pltpu.VMEM((EPC,), jnp.int32),
        pltpu.VMEM((8, KD), jnp.int32),
        pltpu.VMEM((NS, SLC), F32),
        pltpu.VMEM((KD, 128), F32),
        pltpu.VMEM((KD, H), F32),
        pltpu.VMEM((128,), F32),
        pltpu.VMEM_SHARED((QN + 8, H), F32),
        pltpu.SemaphoreType.DMA,
    ],
)


def _gru_body(x_ref, h0_ref, wih_ref, whh_ref, bih_ref, bhh_ref, pw_ref,
              pb_ref, out_ref, h_s, gi_s, hs_s):
    i = pl.program_id(0)
    xb = x_ref[:]
    xe = jnp.where(xb > 0, xb, (jnp.exp(xb) - 1.0))
    xf = xe.reshape(NB * 16, H)
    gi_s[:] = (jnp.dot(xf, wih_ref[:], preferred_element_type=F32)
               + bih_ref[:]).reshape(NB * 8, 2, 3 * GRU_H)

    @pl.when(i == 0)
    def _():
        h_s[:] = h0_ref[:]

    def step(k, _):
        gi = gi_s[k]
        hcur = h_s[:]
        gh = jnp.dot(hcur, whh_ref[:], preferred_element_type=F32) + bhh_ref[:]
        ir, iz, inn = gi[:, :64], gi[:, 64:128], gi[:, 128:]
        hr, hz, hn = gh[:, :64], gh[:, 64:128], gh[:, 128:]
        r = jax.nn.sigmoid(ir + hr)
        zt = jax.nn.sigmoid(iz + hz)
        nn_ = jnp.tanh(inn + r * hn)
        h_s[:] = (1.0 - zt) * nn_ + zt * hcur
        return 0

    def loc(n, _):
        lax.fori_loop(n * 8, n * 8 + 8, step, 0)
        hs_s[n] = h_s[:]
        return 0

    lax.fori_loop(0, NB, loc, 0)
    hs = hs_s[:].reshape(NB * 2, GRU_H)
    out_ref[:] = (jnp.dot(hs, pw_ref[:], preferred_element_type=F32)
                  + pb_ref[:]).reshape(NB, 2, 128)


_gru = pl.pallas_call(
    _gru_body,
    grid=(NPAD // NB,),
    in_specs=[
        pl.BlockSpec((NB, 8, 2, H), lambda i: (i, 0, 0, 0)),
        pl.BlockSpec((2, GRU_H), lambda i: (0, 0)),
        pl.BlockSpec((GRU_H, 3 * GRU_H), lambda i: (0, 0)),
        pl.BlockSpec((GRU_H, 3 * GRU_H), lambda i: (0, 0)),
        pl.BlockSpec((1, 3 * GRU_H), lambda i: (0, 0)),
        pl.BlockSpec((1, 3 * GRU_H), lambda i: (0, 0)),
        pl.BlockSpec((GRU_H, 128), lambda i: (0, 0)),
        pl.BlockSpec((1, 128), lambda i: (0, 0)),
    ],
    out_specs=pl.BlockSpec((NB, 2, 128), lambda i: (i, 0, 0)),
    out_shape=jax.ShapeDtypeStruct((NPAD, 2, 128), F32),
    scratch_shapes=[
        pltpu.VMEM((2, GRU_H), F32),
        pltpu.VMEM((NB * 8, 2, 3 * GRU_H), F32),
        pltpu.VMEM((NB, 2, GRU_H), F32),
    ],
    compiler_params=pltpu.CompilerParams(
        dimension_semantics=("arbitrary",)),
)


_proj1 = _make_proj(128, False)
_proj2 = _make_proj(H, True)


@jax.jit
def kernel(dynamic, h, edge_index, fc1_W, fc1_b, attn1_W, attn1_b, fc2_W,
           fc2_b, attn2_W, attn2_b, gru_W_ih, gru_W_hh, gru_b_ih, gru_b_hh,
           pred_W, pred_b):
    B_, N_, T_, F_ = dynamic.shape
    src = edge_index[0]
    dst = edge_index[1]
    xs = dynamic.transpose(0, 2, 1, 3).reshape(B_ * T_, N_, F_)
    xs = jnp.pad(xs, ((0, 0), (0, NPAD - N_), (0, 0)))

    z1, sa1, da1, mx1 = _proj1(
        xs, fc1_W.T, fc1_b.reshape(1, H), attn1_W[:, :H], attn1_W[:, H:],
        attn1_b.reshape(1, 1))
    ca1 = _shift(da1, mx1)
    g1, _, _ = _gat_sc(z1, sa1.reshape(NG, NPAD), da1.reshape(NG, NPAD),
                       ca1.reshape(NG, NPAD), src, dst)

    z2, sa2, da2, mx2 = _proj2(
        g1, fc2_W.T, fc2_b.reshape(1, H), attn2_W[:, :H], attn2_W[:, H:],
        attn2_b.reshape(1, 1))
    ca2 = _shift(da2, mx2)
    g2, _, _ = _gat_sc(z2, sa2.reshape(NG, NPAD), da2.reshape(NG, NPAD),
                       ca2.reshape(NG, NPAD), src, dst)

    xg = g2.reshape(B_, T_, NPAD, H).transpose(2, 1, 0, 3)

    pw = jnp.zeros((GRU_H, 128), F32).at[:, :pred_W.shape[0]].set(pred_W.T)
    pb = jnp.zeros((1, 128), F32).at[0, :pred_b.shape[0]].set(pred_b)
    outs = _gru(xg, h[0], gru_W_ih.T, gru_W_hh.T,
                gru_b_ih.reshape(1, 3 * GRU_H), gru_b_hh.reshape(1, 3 * GRU_H),
                pw, pb)
    return outs[:N_, :, :pred_W.shape[0]].transpose(1, 0, 2)


# speedup vs baseline: 1.4296x; 1.4296x over previous
import functools

import jax
import jax.numpy as jnp
from jax import lax
from jax.experimental import pallas as pl
from jax.experimental.pallas import tpu as pltpu
from jax.experimental.pallas import tpu_sc as plsc

N_LOC = 10000
NPAD = 10240
E = 160000
NG = 16
H = 64
GRU_H = 64
NC = 2
NS = 16
EPC = E // NS
SLC = NPAD // NS
CH_A = EPC // 16
KD = 80
CH_D = EPC // KD
NQ = 20
QN = NPAD // NQ
NB = 512
F32 = jnp.float32


def _proj_body(apply_elu, x_ref, w_ref, b_ref, as_ref, ad_ref, ab_ref,
               z_ref, sa_ref, da_ref, mx_ref):
    j = pl.program_id(1)
    x = x_ref[0]
    if apply_elu:
        x = jnp.where(x > 0, x, (jnp.exp(x) - 1.0))
    z = jnp.dot(x, w_ref[:], preferred_element_type=F32) + b_ref[:]
    z_ref[0] = jnp.concatenate([z, jnp.zeros((NB, 128 - H), F32)], axis=1)
    sa = jnp.sum(z * as_ref[:], axis=1)
    da = jnp.sum(z * ad_ref[:], axis=1) + ab_ref[0, 0]
    sa_ref[0, 0, 0] = sa
    da_ref[0, 0, 0] = da

    @pl.when(j == 0)
    def _():
        mx_ref[:] = jnp.full((1, 1, 128), -jnp.inf, F32)
    mx_ref[:] = jnp.maximum(mx_ref[:], jnp.max(sa))


def _make_proj(f_in, apply_elu):
    grid = (NG, NPAD // NB)
    return pl.pallas_call(
        functools.partial(_proj_body, apply_elu),
        grid=grid,
        in_specs=[
            pl.BlockSpec((1, NB, f_in), lambda i, j: (i, j, 0)),
            pl.BlockSpec((f_in, H), lambda i, j: (0, 0)),
            pl.BlockSpec((1, H), lambda i, j: (0, 0)),
            pl.BlockSpec((1, H), lambda i, j: (0, 0)),
            pl.BlockSpec((1, H), lambda i, j: (0, 0)),
            pl.BlockSpec((1, 1), lambda i, j: (0, 0)),
        ],
        out_specs=[
            pl.BlockSpec((1, NB, 128), lambda i, j: (i, j, 0)),
            pl.BlockSpec((1, 1, 1, NB), lambda i, j: (i, j, 0, 0)),
            pl.BlockSpec((1, 1, 1, NB), lambda i, j: (i, j, 0, 0)),
            pl.BlockSpec((1, 1, 128), lambda i, j: (i, 0, 0)),
        ],
        out_shape=[
            jax.ShapeDtypeStruct((NG, NPAD, 128), F32),
            jax.ShapeDtypeStruct((NG, NPAD // NB, 1, NB), F32),
            jax.ShapeDtypeStruct((NG, NPAD // NB, 1, NB), F32),
            jax.ShapeDtypeStruct((NG, 1, 128), F32),
        ],
    )


def _shift_body(da_ref, mx_ref, ca_ref):
    u = mx_ref[0, 0, 0] + da_ref[0, 0, 0]
    ca_ref[0, 0, 0] = jnp.where(u >= 0, u, 0.01 * u)


_shift = pl.pallas_call(
    _shift_body,
    grid=(NG, NPAD // NB),
    in_specs=[
        pl.BlockSpec((1, 1, 1, NB), lambda i, j: (i, j, 0, 0)),
        pl.BlockSpec((1, 1, 128), lambda i, j: (i, 0, 0)),
    ],
    out_specs=pl.BlockSpec((1, 1, 1, NB), lambda i, j: (i, j, 0, 0)),
    out_shape=jax.ShapeDtypeStruct((NG, NPAD // NB, 1, NB), F32),
)


def _gat_sc_body(z_hbm, sa_hbm, da_hbm, ca_hbm, src_hbm, dst_hbm, rng_hbm,
                 out_hbm, den_hbm, rden_hbm,
                 sa_v, da_v, ca_v, den_v, rden_v, ex_v, src_v, dst_v, idxq_v,
                 rng_v, red_v, rows_v, rows64_v, w_v,
                 o_sp, sem):
    cid = lax.axis_index("c")
    sid = lax.axis_index("s")
    ebase = sid * EPC
    nbase = sid * SLC
    nslice = pl.ds(nbase, SLC)

    pltpu.sync_copy(src_hbm.at[pl.ds(ebase, EPC)], src_v)
    pltpu.sync_copy(dst_hbm.at[pl.ds(ebase, EPC)], dst_v)
    pltpu.sync_copy(rng_hbm.at[sid], rng_v)

    zero16 = jnp.zeros((16,), F32)

    def per_graph(gl, _):
        g = cid * (NG // NC) + gl

        pltpu.sync_copy(sa_hbm.at[g], sa_v)
        pltpu.sync_copy(da_hbm.at[g], da_v)
        pltpu.sync_copy(ca_hbm.at[g], ca_v)

        def zden(i, _):
            den_v[pl.ds(i * 16, 16)] = zero16
            return 0
        lax.fori_loop(0, NPAD // 16, zden, 0)

        def edges_a(i, _):
            es = pl.ds(i * 16, 16)
            si = src_v[es]
            di = dst_v[es]
            s = plsc.load_gather(sa_v, [si])
            d = plsc.load_gather(da_v, [di])
            c = plsc.load_gather(ca_v, [di])
            u = s + d
            e = jnp.where(u >= 0, u, 0.01 * u)
            ex = jnp.exp(e - c)
            ex_v[es] = ex
            plsc.addupdate_scatter(den_v, [di], ex)
            return 0
        lax.fori_loop(0, CH_A, edges_a, 0)

        pltpu.sync_copy(den_v, den_hbm.at[cid, sid])
        plsc.subcore_barrier()

        for r in range(NS):
            pltpu.sync_copy(den_hbm.at[cid, r].at[nslice], red_v.at[r])

        def redc(c2, _):
            ls = pl.ds(c2 * 16, 16)
            acc = zero16

            def addr(r, a):
                return a + red_v[r, ls]
            acc = lax.fori_loop(0, NS, addr, acc)
            rec = 1.0 / acc
            rden_v[pl.ds(nbase + c2 * 16, 16)] = jnp.where(acc > 0, rec, 1.0)
            return 0
        lax.fori_loop(0, SLC // 16, redc, 0)
        pltpu.sync_copy(rden_v.at[nslice], rden_hbm.at[cid].at[nslice])
        plsc.subcore_barrier()
        pltpu.sync_copy(rden_hbm.at[cid], rden_v)

        def zrow(k, _):
            for q in range(4):
                rows64_v[k, pl.ds(q * 16, 16)] = zero16
            return 0

        for q in range(NQ):
            qb = q * QN
            lax.fori_loop(0, KD, zrow, 0)
            pltpu.sync_copy(rows64_v.at[pl.ds(0, QN // NS)],
                            o_sp.at[pl.ds(sid * (QN // NS), QN // NS)])
            plsc.subcore_barrier()

            def edges_d(j, _):
                base = j * KD
                for m in range(KD // 16):
                    es = pl.ds(base + m * 16, 16)
                    di = dst_v[es]
                    rd = plsc.load_gather(rden_v, [di])
                    w_v[pl.ds(m * 16, 16)] = ex_v[es] * rd
                    inr = (di >= qb) & (di < qb + QN)
                    idxq_v[0, pl.ds(m * 16, 16)] = jnp.where(inr, di - qb, QN)
                pltpu.async_copy(z_hbm.at[g].at[src_v.at[pl.ds(base, KD)]],
                                 rows_v, sem).wait()

                def srow(k, _):
                    wk = plsc.load_gather(
                        w_v, [jnp.full((16,), 0, jnp.int32) + k])
                    for qq in range(4):
                        ls = pl.ds(qq * 16, 16)
                        rows64_v[k, ls] = rows_v[k, ls] * wk
                    return 0
                lax.fori_loop(0, KD, srow, 0)
                pltpu.sync_copy(rows64_v, o_sp.at[idxq_v.at[0]], add=True)
                return 0
            rv = rng_v[q]
            lax.fori_loop(rv[0], rv[1], edges_d, 0)
            plsc.subcore_barrier()

            pltpu.sync_copy(o_sp.at[pl.ds(sid * (QN // NS), QN // NS)],
                            rows64_v.at[pl.ds(0, QN // NS)])
            pltpu.sync_copy(rows64_v.at[pl.ds(0, QN // NS)],
                            out_hbm.at[g].at[pl.ds(qb + sid * (QN // NS),
                                                   QN // NS)])
            plsc.subcore_barrier()
        return 0

    lax.fori_loop(0, NG // NC, per_graph, 0)


_gat_sc = pl.kernel(
    _gat_sc_body,
    out_type=(jax.ShapeDtypeStruct((NG, NPAD, H), F32),
              jax.ShapeDtypeStruct((NC, NS, NPAD), F32),
              jax.ShapeDtypeStruct((NC, NPAD), F32)),
    mesh=plsc.VectorSubcoreMesh(core_axis_name="c", subcore_axis_name="s"),
    compiler_params=pltpu.CompilerParams(needs_layout_passes=False),
    scratch_types=[
        pltpu.VMEM((NPAD,), F32),
        pltpu.VMEM((NPAD,), F32),
        pltpu.VMEM((NPAD,), F32),
        pltpu.VMEM((NPAD,), F32),
        pltpu.VMEM((NPAD,), F32),
        pltpu.VMEM((EPC,), F32),
        pltpu.VMEM((EPC,), jnp.int32),
        pltpu.VMEM((EPC,), jnp.int32),
        pltpu.VMEM((8, KD), jnp.int32),
        pltpu.VMEM((NQ, 16), jnp.int32),
        pltpu.VMEM((NS, SLC), F32),
        pltpu.VMEM((KD, 128), F32),
        pltpu.VMEM((KD, H), F32),
        pltpu.VMEM((128,), F32),
        pltpu.VMEM_SHARED((QN + 8, H), F32),
        pltpu.SemaphoreType.DMA,
    ],
)


def _gru_body(x_ref, h0_ref, wih_ref, whh_ref, bih_ref, bhh_ref, pw_ref,
              pb_ref, out_ref, h_s, gi_s, hs_s):
    i = pl.program_id(0)
    xb = x_ref[:]
    xe = jnp.where(xb > 0, xb, (jnp.exp(xb) - 1.0))
    xf = xe.reshape(NB * 16, H)
    gi_s[:] = (jnp.dot(xf, wih_ref[:], preferred_element_type=F32)
               + bih_ref[:]).reshape(NB * 8, 2, 3 * GRU_H)

    @pl.when(i == 0)
    def _():
        h_s[:] = h0_ref[:]

    def step(k, _):
        gi = gi_s[k]
        hcur = h_s[:]
        gh = jnp.dot(hcur, whh_ref[:], preferred_element_type=F32) + bhh_ref[:]
        ir, iz, inn = gi[:, :64], gi[:, 64:128], gi[:, 128:]
        hr, hz, hn = gh[:, :64], gh[:, 64:128], gh[:, 128:]
        r = jax.nn.sigmoid(ir + hr)
        zt = jax.nn.sigmoid(iz + hz)
        nn_ = jnp.tanh(inn + r * hn)
        h_s[:] = (1.0 - zt) * nn_ + zt * hcur
        return 0

    def loc(n, _):
        lax.fori_loop(n * 8, n * 8 + 8, step, 0)
        hs_s[n] = h_s[:]
        return 0

    lax.fori_loop(0, NB, loc, 0)
    hs = hs_s[:].reshape(NB * 2, GRU_H)
    out_ref[:] = (jnp.dot(hs, pw_ref[:], preferred_element_type=F32)
                  + pb_ref[:]).reshape(NB, 2, 128)


_gru = pl.pallas_call(
    _gru_body,
    grid=(NPAD // NB,),
    in_specs=[
        pl.BlockSpec((NB, 8, 2, H), lambda i: (i, 0, 0, 0)),
        pl.BlockSpec((2, GRU_H), lambda i: (0, 0)),
        pl.BlockSpec((GRU_H, 3 * GRU_H), lambda i: (0, 0)),
        pl.BlockSpec((GRU_H, 3 * GRU_H), lambda i: (0, 0)),
        pl.BlockSpec((1, 3 * GRU_H), lambda i: (0, 0)),
        pl.BlockSpec((1, 3 * GRU_H), lambda i: (0, 0)),
        pl.BlockSpec((GRU_H, 128), lambda i: (0, 0)),
        pl.BlockSpec((1, 128), lambda i: (0, 0)),
    ],
    out_specs=pl.BlockSpec((NB, 2, 128), lambda i: (i, 0, 0)),
    out_shape=jax.ShapeDtypeStruct((NPAD, 2, 128), F32),
    scratch_shapes=[
        pltpu.VMEM((2, GRU_H), F32),
        pltpu.VMEM((NB * 8, 2, 3 * GRU_H), F32),
        pltpu.VMEM((NB, 2, GRU_H), F32),
    ],
    compiler_params=pltpu.CompilerParams(
        dimension_semantics=("arbitrary",)),
)


_proj1 = _make_proj(128, False)
_proj2 = _make_proj(H, True)


@jax.jit
def kernel(dynamic, h, edge_index, fc1_W, fc1_b, attn1_W, attn1_b, fc2_W,
           fc2_b, attn2_W, attn2_b, gru_W_ih, gru_W_hh, gru_b_ih, gru_b_hh,
           pred_W, pred_b):
    B_, N_, T_, F_ = dynamic.shape
    order = jnp.argsort(edge_index[1])
    src = edge_index[0][order]
    dst = edge_index[1][order]
    qb = jnp.searchsorted(dst, jnp.arange(NQ + 1) * QN).astype(jnp.int32)
    sbase = jnp.arange(NS, dtype=jnp.int32)[:, None] * EPC
    lo_c = jnp.clip(qb[None, :-1] - sbase, 0, EPC) // KD
    hi_c = -((-jnp.clip(qb[None, 1:] - sbase, 0, EPC)) // KD)
    rng = jnp.zeros((NS, NQ, 16), jnp.int32)
    rng = rng.at[:, :, 0].set(lo_c).at[:, :, 1].set(hi_c)
    xs = dynamic.transpose(0, 2, 1, 3).reshape(B_ * T_, N_, F_)
    xs = jnp.pad(xs, ((0, 0), (0, NPAD - N_), (0, 0)))

    z1, sa1, da1, mx1 = _proj1(
        xs, fc1_W.T, fc1_b.reshape(1, H), attn1_W[:, :H], attn1_W[:, H:],
        attn1_b.reshape(1, 1))
    ca1 = _shift(da1, mx1)
    g1, _, _ = _gat_sc(z1, sa1.reshape(NG, NPAD), da1.reshape(NG, NPAD),
                       ca1.reshape(NG, NPAD), src, dst, rng)

    z2, sa2, da2, mx2 = _proj2(
        g1, fc2_W.T, fc2_b.reshape(1, H), attn2_W[:, :H], attn2_W[:, H:],
        attn2_b.reshape(1, 1))
    ca2 = _shift(da2, mx2)
    g2, _, _ = _gat_sc(z2, sa2.reshape(NG, NPAD), da2.reshape(NG, NPAD),
                       ca2.reshape(NG, NPAD), src, dst, rng)

    xg = g2.reshape(B_, T_, NPAD, H).transpose(2, 1, 0, 3)

    pw = jnp.zeros((GRU_H, 128), F32).at[:, :pred_W.shape[0]].set(pred_W.T)
    pb = jnp.zeros((1, 128), F32).at[0, :pred_b.shape[0]].set(pred_b)
    outs = _gru(xg, h[0], gru_W_ih.T, gru_W_hh.T,
                gru_b_ih.reshape(1, 3 * GRU_H), gru_b_hh.reshape(1, 3 * GRU_H),
                pw, pb)
    return outs[:N_, :, :pred_W.shape[0]].transpose(1, 0, 2)
